# baseline (speedup 1.0000x reference)
import functools

import jax
import jax.numpy as jnp
from jax import lax
from jax.experimental import pallas as pl
from jax.experimental.pallas import tpu as pltpu
from jax.experimental.pallas import tpu_sc as plsc

_NUM_E = 1000000
_H = 64
_BATCH = 16384
_NW = 32
_BPW = _BATCH // _NW
_CHUNK = 128
_NCHUNK = _BPW // _CHUNK


def _gate_body(table_hbm, idx_hbm, half_hbm, w_hbm, b_hbm, out_hbm,
               idx_v, half_v, rows_v, w_v, b_v, t_v, out_v, sem):
    wid = lax.axis_index("s") * 2 + lax.axis_index("c")
    base = wid * _BPW

    for j in range(_NCHUNK):
        pltpu.sync_copy(idx_hbm.at[pl.ds(base + j * _CHUNK, _CHUNK)],
                        idx_v.at[j])
    pltpu.sync_copy(half_hbm.at[pl.ds(base, _BPW)], half_v)
    pltpu.sync_copy(w_hbm, w_v)
    pltpu.sync_copy(b_hbm, b_v)

    copies = [
        pltpu.async_copy(
            table_hbm.at[idx_v.at[j]],
            rows_v.at[pl.ds(j * _CHUNK, _CHUNK)],
            sem,
        )
        for j in range(_NCHUNK)
    ]
    for c in copies:
        c.wait()

    wv = [w_v[pl.ds(16 * c, 16)] for c in range(4)]
    bv = b_v[...]
    lanes = lax.iota(jnp.int32, 16)
    zv = jnp.zeros((16,), jnp.float32)

    for r in range(16):
        t_v[pl.ds(r * 48, 16)] = zv
        t_v[pl.ds(r * 48 + 32, 16)] = zv

    def body(g, carry):
        acc = zv
        hv = half_v[pl.ds(g * 16, 16)] * 64
        for r in range(16):
            i = g * 16 + r
            hoff = hv[r]
            p = rows_v[i, pl.ds(hoff, 16)] * wv[0]
            p += rows_v[i, pl.ds(hoff + 16, 16)] * wv[1]
            p += rows_v[i, pl.ds(hoff + 32, 16)] * wv[2]
            p += rows_v[i, pl.ds(hoff + 48, 16)] * wv[3]
            s = p
            mid = r * 48 + 16
            for d in (8, 4, 2, 1):
                t_v[pl.ds(mid, 16)] = s
                off = -d if (r & d) else d
                s = s + t_v[pl.ds(mid + off, 16)]
            acc = jnp.where(lanes == r, s, acc)
        x = acc + bv
        out_v[pl.ds(g * 16, 16)] = 1.0 / (1.0 + jnp.exp(-x))
        return carry

    lax.fori_loop(0, _BPW // 16, body, 0)

    pltpu.sync_copy(out_v, out_hbm.at[pl.ds(base, _BPW)])


@jax.jit
def _gate_sc(idx, half, table2, w_flat, b_vec):
    mesh = plsc.VectorSubcoreMesh(core_axis_name="c", subcore_axis_name="s")
    f = functools.partial(
        pl.kernel,
        mesh=mesh,
        out_type=jax.ShapeDtypeStruct((_BATCH,), jnp.float32),
        scratch_types=[
            pltpu.VMEM((_NCHUNK, _CHUNK), jnp.int32),
            pltpu.VMEM((_BPW,), jnp.int32),
            pltpu.VMEM((_BPW, 2 * _H), jnp.float32),
            pltpu.VMEM((_H,), jnp.float32),
            pltpu.VMEM((16,), jnp.float32),
            pltpu.VMEM((16 * 48,), jnp.float32),
            pltpu.VMEM((_BPW,), jnp.float32),
            pltpu.SemaphoreType.DMA,
        ],
    )(_gate_body)
    return f(table2, idx, half, w_flat, b_vec)


def kernel(X, Y, gate_theta, W, b):
    idx = jax.lax.shift_right_logical(X, 1)
    half = jax.lax.bitwise_and(X, 1)
    table2 = gate_theta.reshape(_NUM_E // 2, 2 * _H)
    w_flat = W.reshape(_H)
    b_vec = jnp.broadcast_to(b.reshape(()), (16,)).astype(jnp.float32)
    out = _gate_sc(idx, half, table2, w_flat, b_vec)
    return out.reshape(_BATCH, 1)


# speedup vs baseline: 1.7085x; 1.7085x over previous
import functools

import jax
import jax.numpy as jnp
from jax import lax
from jax.experimental import pallas as pl
from jax.experimental.pallas import tpu as pltpu
from jax.experimental.pallas import tpu_sc as plsc

_NUM_E = 1000000
_H = 64
_BATCH = 16384
_NW = 32
_BPW = _BATCH // _NW


def _gate_body(table_hbm, idx_hbm, w_hbm, b_hbm, out_hbm,
               idx_v, rows_v, w_v, b_v, t_v, out_v, sem):
    wid = lax.axis_index("s") * 2 + lax.axis_index("c")
    base = wid * _BPW

    pltpu.sync_copy(idx_hbm.at[pl.ds(base, _BPW)], idx_v)
    pltpu.sync_copy(w_hbm, w_v)
    pltpu.sync_copy(b_hbm, b_v)

    for c in range(_BPW // 16):
        iv = idx_v[pl.ds(c * 16, 16)]
        for r in range(16):
            i = c * 16 + r
            pltpu.async_copy(
                table_hbm.at[pl.ds(iv[r], 1)],
                rows_v.at[pl.ds(i, 1)],
                sem,
            )
    pltpu.make_async_copy(table_hbm.at[pl.ds(0, _BPW)], rows_v, sem).wait()

    wv = [w_v[pl.ds(16 * c, 16)] for c in range(4)]
    bv = b_v[...]
    lanes = lax.iota(jnp.int32, 16)
    zv = jnp.zeros((16,), jnp.float32)

    for r in range(16):
        t_v[pl.ds(r * 48, 16)] = zv
        t_v[pl.ds(r * 48 + 32, 16)] = zv

    def body(g, carry):
        acc = zv
        for r in range(16):
            i = g * 16 + r
            p = rows_v[i, pl.ds(0, 16)] * wv[0]
            p += rows_v[i, pl.ds(16, 16)] * wv[1]
            p += rows_v[i, pl.ds(32, 16)] * wv[2]
            p += rows_v[i, pl.ds(48, 16)] * wv[3]
            s = p
            mid = r * 48 + 16
            for d in (8, 4, 2, 1):
                t_v[pl.ds(mid, 16)] = s
                off = -d if (r & d) else d
                s = s + t_v[pl.ds(mid + off, 16)]
            acc = jnp.where(lanes == r, s, acc)
        x = acc + bv
        out_v[pl.ds(g * 16, 16)] = 1.0 / (1.0 + jnp.exp(-x))
        return carry

    lax.fori_loop(0, _BPW // 16, body, 0)

    pltpu.sync_copy(out_v, out_hbm.at[pl.ds(base, _BPW)])


@jax.jit
def _gate_sc(idx, table, w_flat, b_vec):
    mesh = plsc.VectorSubcoreMesh(core_axis_name="c", subcore_axis_name="s")
    f = functools.partial(
        pl.kernel,
        mesh=mesh,
        out_type=jax.ShapeDtypeStruct((_BATCH,), jnp.float32),
        scratch_types=[
            pltpu.VMEM((_BPW,), jnp.int32),
            pltpu.VMEM((_BPW, _H), jnp.float32),
            pltpu.VMEM((_H,), jnp.float32),
            pltpu.VMEM((16,), jnp.float32),
            pltpu.VMEM((16 * 48,), jnp.float32),
            pltpu.VMEM((_BPW,), jnp.float32),
            pltpu.SemaphoreType.DMA,
        ],
    )(_gate_body)
    return f(table, idx, w_flat, b_vec)


def kernel(X, Y, gate_theta, W, b):
    w_flat = W.reshape(_H)
    b_vec = jnp.broadcast_to(b.reshape(()), (16,)).astype(jnp.float32)
    out = _gate_sc(X, gate_theta, w_flat, b_vec)
    return out.reshape(_BATCH, 1)


# speedup vs baseline: 1.9009x; 1.1126x over previous
import functools

import jax
import jax.numpy as jnp
from jax import lax
from jax.experimental import pallas as pl
from jax.experimental.pallas import tpu as pltpu
from jax.experimental.pallas import tpu_sc as plsc

_NUM_E = 1000000
_H = 64
_BATCH = 16384
_NW = 32
_BPW = _BATCH // _NW
_CHUNK = 128
_NCHUNK = _BPW // _CHUNK

_BLK = 2048
_NBLK = (_NUM_E + _BLK - 1) // _BLK
_YPAD = _NBLK * _BLK


def _matvec_body(w_ref, b_ref, tbl_ref, y_ref):
    x = jnp.dot(w_ref[...], tbl_ref[...]) + b_ref[0, 0]
    y_ref[...] = (1.0 / (1.0 + jnp.exp(-x))).reshape(_BLK)


@jax.jit
def _gate_all_tc(tableT, w_row, b2):
    return pl.pallas_call(
        _matvec_body,
        grid=(_NBLK,),
        in_specs=[
            pl.BlockSpec((1, _H), lambda j: (0, 0)),
            pl.BlockSpec((1, 1), lambda j: (0, 0)),
            pl.BlockSpec((_H, _BLK), lambda j: (0, j)),
        ],
        out_specs=pl.BlockSpec((_BLK,), lambda j: (j,)),
        out_shape=jax.ShapeDtypeStruct((_YPAD,), jnp.float32),
    )(w_row, b2, tableT)


def _gather_body(y_hbm, idx_hbm, out_hbm, idx_v, g_v, sem):
    wid = lax.axis_index("s") * 2 + lax.axis_index("c")

    pltpu.sync_copy(idx_hbm.at[wid], idx_v)
    copies = [
        pltpu.async_copy(
            y_hbm.at[idx_v.at[j]],
            g_v.at[j],
            sem,
        )
        for j in range(_NCHUNK)
    ]
    for c in copies:
        c.wait()
    pltpu.sync_copy(g_v, out_hbm.at[wid])


@jax.jit
def _gather_sc(y1d, idx):
    mesh = plsc.VectorSubcoreMesh(core_axis_name="c", subcore_axis_name="s")
    f = functools.partial(
        pl.kernel,
        mesh=mesh,
        out_type=jax.ShapeDtypeStruct((_NW, _NCHUNK, _CHUNK), jnp.float32),
        scratch_types=[
            pltpu.VMEM((_NCHUNK, _CHUNK), jnp.int32),
            pltpu.VMEM((_NCHUNK, _CHUNK), jnp.float32),
            pltpu.SemaphoreType.DMA,
        ],
    )(_gather_body)
    return f(y1d, idx)


def kernel(X, Y, gate_theta, W, b):
    w_row = W.reshape(1, _H)
    b2 = b.reshape(1, 1)
    y = _gate_all_tc(gate_theta.T, w_row, b2)
    idx = X.reshape(_NW, _NCHUNK, _CHUNK)
    return _gather_sc(y, idx).reshape(_BATCH, 1)


# speedup vs baseline: 6.6123x; 3.4785x over previous
import functools

import jax
import jax.numpy as jnp
from jax import lax
from jax.experimental import pallas as pl
from jax.experimental.pallas import tpu as pltpu
from jax.experimental.pallas import tpu_sc as plsc

_NUM_E = 1000000
_H = 64
_BATCH = 16384
_NW = 32
_BPW = _BATCH // _NW
_CHUNK = 128
_NCHUNK = _BPW // _CHUNK

_BLK = 32768
_NBLK = (_NUM_E + _BLK - 1) // _BLK
_YPAD = _NBLK * _BLK


def _matvec_body(w_ref, b_ref, tbl_ref, y_ref):
    x = jnp.dot(w_ref[...], tbl_ref[...]) + b_ref[0, 0]
    y_ref[...] = (1.0 / (1.0 + jnp.exp(-x))).reshape(_BLK)


@jax.jit
def _gate_all_tc(tableT, w_row, b2):
    return pl.pallas_call(
        _matvec_body,
        grid=(_NBLK,),
        in_specs=[
            pl.BlockSpec((1, _H), lambda j: (0, 0)),
            pl.BlockSpec((1, 1), lambda j: (0, 0)),
            pl.BlockSpec((_H, _BLK), lambda j: (0, j)),
        ],
        out_specs=pl.BlockSpec((_BLK,), lambda j: (j,)),
        out_shape=jax.ShapeDtypeStruct((_YPAD,), jnp.float32),
    )(w_row, b2, tableT)


def _gather_body(y_hbm, idx_hbm, out_hbm, idx_v, g_v, sem):
    wid = lax.axis_index("s") * 2 + lax.axis_index("c")

    pltpu.sync_copy(idx_hbm.at[wid], idx_v)
    copies = [
        pltpu.async_copy(
            y_hbm.at[idx_v.at[j]],
            g_v.at[j],
            sem,
        )
        for j in range(_NCHUNK)
    ]
    for c in copies:
        c.wait()
    pltpu.sync_copy(g_v, out_hbm.at[wid])


@jax.jit
def _gather_sc(y1d, idx):
    mesh = plsc.VectorSubcoreMesh(core_axis_name="c", subcore_axis_name="s")
    f = functools.partial(
        pl.kernel,
        mesh=mesh,
        out_type=jax.ShapeDtypeStruct((_NW, _NCHUNK, _CHUNK), jnp.float32),
        scratch_types=[
            pltpu.VMEM((_NCHUNK, _CHUNK), jnp.int32),
            pltpu.VMEM((_NCHUNK, _CHUNK), jnp.float32),
            pltpu.SemaphoreType.DMA,
        ],
    )(_gather_body)
    return f(y1d, idx)


def kernel(X, Y, gate_theta, W, b):
    w_row = W.reshape(1, _H)
    b2 = b.reshape(1, 1)
    y = _gate_all_tc(gate_theta.T, w_row, b2)
    idx = X.reshape(_NW, _NCHUNK, _CHUNK)
    return _gather_sc(y, idx).reshape(_BATCH, 1)
